# baseline (speedup 1.0000x reference)
import jax
import jax.numpy as jnp
from jax import lax
from jax.experimental import pallas as pl
from jax.experimental.pallas import tpu as pltpu
from jax.experimental.pallas import tpu_sc as plsc

B = 16384
EMBED = 16
H = 64

_info = plsc.get_sparse_core_info()
_NC, _NS = _info.num_cores, _info.num_subcores
_NW = _NC * _NS
_HALF = B // 2
_BPW = _HALF // _NW
_GRP = 16


def _gather_body(idx_hbm, tableT_hbm, out_hbm, idx_v, e_v, sem, *slots):
    wid = lax.axis_index("s") * _NC + lax.axis_index("c")
    base = wid * _BPW
    pltpu.sync_copy(idx_hbm.at[pl.ds(base, _BPW)], idx_v)
    lanes = lax.iota(jnp.int32, 16)

    def group_step(g, carry):
        p0 = g * _GRP
        v = idx_v[pl.ds(p0, _GRP)]
        copies = []
        for j in range(_GRP):
            k = v[j]
            lane_base = pl.multiple_of(
                lax.shift_left(lax.shift_right_logical(k, 7), 7), 128
            )
            copies.append(
                pltpu.async_copy(
                    tableT_hbm.at[:, pl.ds(lane_base, 128)], slots[j], sem
                )
            )
        for j in range(_GRP):
            copies[j].wait()
            col = jnp.bitwise_and(v[j], 127)
            vals = plsc.load_gather(
                slots[j], [lanes, jnp.full((16,), 0, jnp.int32) + col]
            )
            e_v[p0 + j, :] = vals
        return carry

    lax.fori_loop(0, _BPW // _GRP, group_step, 0)
    pltpu.sync_copy(e_v, out_hbm.at[pl.ds(base, _BPW), :])


_sc_gather = pl.kernel(
    _gather_body,
    mesh=plsc.VectorSubcoreMesh(core_axis_name="c", subcore_axis_name="s"),
    out_type=jax.ShapeDtypeStruct((_HALF, EMBED), jnp.float32),
    scratch_types=[
        pltpu.VMEM((_BPW,), jnp.int32),
        pltpu.VMEM((_BPW, EMBED), jnp.float32),
        pltpu.SemaphoreType.DMA,
    ] + [pltpu.VMEM((EMBED, 128), jnp.float32) for _ in range(_GRP)],
    compiler_params=pltpu.CompilerParams(needs_layout_passes=False),
)

_BLK = 4096
_MLP_GRID = _HALF // _BLK


def _mlp_body(e_ref, W1_ref, b1_ref, W2_ref, b2_ref, out_ref):
    h = jnp.dot(e_ref[...], W1_ref[...], preferred_element_type=jnp.float32)
    h = jnp.maximum(h + b1_ref[...], 0.0)
    o = jnp.dot(h, W2_ref[...], preferred_element_type=jnp.float32)
    out_ref[...] = o + b2_ref[...]


_tc_mlp = pl.pallas_call(
    _mlp_body,
    grid=(_MLP_GRID,),
    in_specs=[
        pl.BlockSpec((_BLK, EMBED), lambda i: (i, 0)),
        pl.BlockSpec((EMBED, H), lambda i: (0, 0)),
        pl.BlockSpec((1, H), lambda i: (0, 0)),
        pl.BlockSpec((H, 1), lambda i: (0, 0)),
        pl.BlockSpec((1, 1), lambda i: (0, 0)),
    ],
    out_specs=pl.BlockSpec((_BLK, 1), lambda i: (i, 0)),
    out_shape=jax.ShapeDtypeStruct((_HALF, 1), jnp.float32),
)


@jax.jit
def kernel(x, table, W1, b1, W2, b2):
    idx = x.astype(jnp.int32).reshape(B)
    tT = table.T
    b1r, b2r = b1.reshape(1, H), b2.reshape(1, 1)
    e0 = _sc_gather(idx[:_HALF], tT)
    e1 = _sc_gather(idx[_HALF:], tT)
    o0 = _tc_mlp(e0, W1, b1r, W2, b2r)
    o1 = _tc_mlp(e1, W1, b1r, W2, b2r)
    return jnp.concatenate([o0, o1], axis=0).reshape(B)
